# baseline (speedup 1.0000x reference)
import functools

import jax
import jax.numpy as jnp
from jax import lax
from jax.experimental import pallas as pl
from jax.experimental.pallas import tpu as pltpu
from jax.experimental.pallas import tpu_sc as plsc

K = 16
PPAD = 16


def _proj_kernel(q_ref, k_ref, v_ref, pp_ref, wq_ref, wk_ref, wv_ref, t_ref):
    f32 = jnp.float32
    d = (jnp.dot(q_ref[...], wq_ref[...], preferred_element_type=f32)
         - jnp.dot(k_ref[...], wk_ref[...], preferred_element_type=f32))
    v = jnp.dot(v_ref[...], wv_ref[...], preferred_element_type=f32)
    t_ref[...] = jnp.concatenate([d, v, pp_ref[...]], axis=1)


def _knn_kernel(crows_ref, ccols_ref, idx_ref, *, n, rb, k, boffset):
    cr = crows_ref[...]
    ca = ccols_ref[...]
    crb = cr.astype(jnp.bfloat16).astype(jnp.float32)
    cab = ca.astype(jnp.bfloat16).astype(jnp.float32)
    dot = lax.dot_general(crb, cab, (((1,), (0,)), ((), ())),
                          precision=lax.Precision.HIGHEST,
                          preferred_element_type=jnp.float32)
    d2r = jnp.sum(cr * cr, axis=1, keepdims=True)
    d2a = jnp.sum(ca * ca, axis=0, keepdims=True)
    dist = d2r + d2a - 2.0 * dot
    dshift = dist + 0.5
    seg = 512
    g_iota = lax.broadcasted_iota(jnp.int32, (rb, n), 1)
    vclass = lax.shift_right_logical(g_iota, 9)
    c = lax.bitcast_convert_type(
        (lax.bitcast_convert_type(dshift, jnp.int32) & jnp.int32(~0x7)) | vclass,
        jnp.float32)
    nseg = n // seg
    iotaseg = lax.broadcasted_iota(jnp.int32, (rb, seg), 1).astype(jnp.float32)
    removed = jnp.float32(jnp.inf)
    mprev = jnp.zeros((rb, 1), dtype=jnp.float32)
    cols = []
    for _ in range(k):
        s = jnp.where(c[:, 0:seg] > mprev, c[:, 0:seg], removed)
        for j in range(1, nseg):
            cj = c[:, j * seg:(j + 1) * seg]
            s = jnp.minimum(s, jnp.where(cj > mprev, cj, removed))
        m = jnp.min(s, axis=1, keepdims=True)
        lane = jnp.min(jnp.where(s == m, iotaseg, jnp.float32(seg)),
                       axis=1, keepdims=True)
        mb = lax.bitcast_convert_type(m, jnp.int32)
        am = ((mb & jnp.int32(7)) << 9) | lane.astype(jnp.int32)
        cols.append(am)
        mprev = m
    idxf = jnp.concatenate(cols, axis=1)
    idx_ref[...] = idxf + boffset


def _mlp_kernel(g_ref, pr_ref, p1_ref, pb1_ref, p2_ref, pb2_ref,
                a1_ref, ab1_ref, a2_ref, ab2_ref, wo_ref, bo_ref, out_ref, *, rb, k):
    f32 = jnp.float32
    g = g_ref[...]
    inner_w = p2_ref.shape[1]
    gd = g[:, 0:inner_w]
    gv = g[:, inner_w:2 * inner_w]
    gp = g[:, 2 * inner_w:2 * inner_w + PPAD]
    pc = pr_ref[...]
    rel = gp - jnp.broadcast_to(pc[:, None, :], (rb, k, PPAD)).reshape(rb * k, PPAD)
    h1 = jnp.maximum(jnp.dot(rel, p1_ref[...], preferred_element_type=f32)
                     + pb1_ref[...], 0.0)
    rpe = jnp.dot(h1, p2_ref[...], preferred_element_type=f32) + pb2_ref[...]
    s = jnp.dot(gd + rpe, a1_ref[...], preferred_element_type=f32) + ab1_ref[...]
    s = jnp.maximum(s, 0.0)
    sim = jnp.dot(s, a2_ref[...], preferred_element_type=f32) + ab2_ref[...]
    e = jnp.exp(sim)
    p = e / jnp.sum(e, axis=-1, keepdims=True)
    vg = gv + rpe
    inner = sim.shape[-1]
    p3 = p.reshape(rb, k, inner)
    vg3 = vg.reshape(rb, k, inner)
    num = jnp.sum(p3 * vg3, axis=1)
    ssq = jnp.sum(p3 * p3, axis=1)
    agg = num / jnp.maximum(jnp.sqrt(ssq), 1e-12)
    out_ref[...] = jnp.dot(agg, wo_ref[...], preferred_element_type=f32) + bo_ref[...]


def _make_gather(nrows, width):
    info = plsc.get_sparse_core_info()
    nc, ns = info.num_cores, info.num_subcores
    nw = nc * ns
    chunk = 128
    per_w = nrows // nw
    nchunk = per_w // chunk
    f32 = jnp.float32
    mesh = plsc.VectorSubcoreMesh(core_axis_name="c", subcore_axis_name="s")

    @functools.partial(
        pl.kernel, mesh=mesh,
        out_type=jax.ShapeDtypeStruct((nrows, width), f32),
        scratch_types=[pltpu.VMEM((chunk,), jnp.int32),
                       pltpu.VMEM((chunk, width), f32),
                       pltpu.SemaphoreType.DMA])
    def gather(tab, idx_hbm, out, idxv, buf, s_g):
        wid = lax.axis_index("s") * nc + lax.axis_index("c")
        base = wid * per_w

        def body(c, carry):
            off = base + c * chunk
            pltpu.sync_copy(idx_hbm.at[pl.ds(off, chunk)], idxv)
            pltpu.async_copy(tab.at[idxv], buf, s_g).wait()
            pltpu.sync_copy(buf, out.at[pl.ds(off, chunk)])
            return carry

        lax.fori_loop(0, nchunk, body, 0)

    return gather


def kernel(query, key, value, canonical, Wq, Wk, Wv, Wo, bo, P1, pb1, P2, pb2, A1, ab1, A2, ab2):
    bs, n, emb = query.shape
    inner = Wq.shape[0]
    ah = A1.shape[0]
    ph = P1.shape[0]
    k = K
    bn = bs * n
    f32 = jnp.float32

    qf = query.reshape(bn, emb)
    kf = key.reshape(bn, emb)
    vf = value.reshape(bn, emb)
    cf = canonical.reshape(bn, 3)

    width = 2 * inner + emb
    pospad = jnp.pad(cf, ((0, 0), (0, emb - 3)))
    rb1 = 512
    tab = pl.pallas_call(
        _proj_kernel,
        grid=(bn // rb1,),
        in_specs=[pl.BlockSpec((rb1, emb), lambda i: (i, 0))] * 4
        + [pl.BlockSpec((emb, inner), lambda i: (0, 0))] * 3,
        out_specs=pl.BlockSpec((rb1, width), lambda i: (i, 0)),
        out_shape=jax.ShapeDtypeStruct((bn, width), f32),
    )(qf, kf, vf, pospad, Wq.T, Wk.T, Wv.T)

    rb2 = 128
    nb2 = n // rb2
    ct = jnp.transpose(canonical, (0, 2, 1))
    pos16 = jnp.pad(cf, ((0, 0), (0, PPAD - 3)))
    p1ext = jnp.pad(P1.T, ((0, PPAD - 3), (0, 0)))
    rb4 = 128
    rows4 = rb4 * k
    gather_fn = _make_gather(n * k, width)

    outs = []
    for b in range(bs):
        idx_b = pl.pallas_call(
            functools.partial(_knn_kernel, n=n, rb=rb2, k=k, boffset=b * n),
            grid=(nb2,),
            in_specs=[pl.BlockSpec((rb2, 3), lambda i: (i, 0)),
                      pl.BlockSpec((3, n), lambda i: (0, 0))],
            out_specs=pl.BlockSpec((rb2, k), lambda i: (i, 0)),
            out_shape=jax.ShapeDtypeStruct((n, k), jnp.int32),
        )(canonical[b], ct[b])

        g_b = gather_fn(tab, idx_b.reshape(n * k))

        out_b = pl.pallas_call(
            functools.partial(_mlp_kernel, rb=rb4, k=k),
            grid=(n // rb4,),
            in_specs=[
                pl.BlockSpec((rows4, width), lambda i: (i, 0)),
                pl.BlockSpec((rb4, PPAD), lambda i: (i, 0)),
                pl.BlockSpec((PPAD, ph), lambda i: (0, 0)),
                pl.BlockSpec((1, ph), lambda i: (0, 0)),
                pl.BlockSpec((ph, inner), lambda i: (0, 0)),
                pl.BlockSpec((1, inner), lambda i: (0, 0)),
                pl.BlockSpec((inner, ah), lambda i: (0, 0)),
                pl.BlockSpec((1, ah), lambda i: (0, 0)),
                pl.BlockSpec((ah, inner), lambda i: (0, 0)),
                pl.BlockSpec((1, inner), lambda i: (0, 0)),
                pl.BlockSpec((inner, emb), lambda i: (0, 0)),
                pl.BlockSpec((1, emb), lambda i: (0, 0)),
            ],
            out_specs=pl.BlockSpec((rb4, emb), lambda i: (i, 0)),
            out_shape=jax.ShapeDtypeStruct((n, emb), f32),
        )(g_b, pos16[b * n:(b + 1) * n], p1ext, pb1.reshape(1, ph), P2.T, pb2.reshape(1, inner),
          A1.T, ab1.reshape(1, ah), A2.T, ab2.reshape(1, inner), Wo.T,
          bo.reshape(1, emb))
        outs.append(out_b)

    return jnp.stack(outs).reshape(bs, n, emb)


# speedup vs baseline: 1.0197x; 1.0197x over previous
import functools

import jax
import jax.numpy as jnp
from jax import lax
from jax.experimental import pallas as pl
from jax.experimental.pallas import tpu as pltpu
from jax.experimental.pallas import tpu_sc as plsc

K = 16
PPAD = 16


def _proj_kernel(q_ref, k_ref, v_ref, pp_ref, wq_ref, wk_ref, wv_ref, t_ref):
    f32 = jnp.float32
    d = (jnp.dot(q_ref[...], wq_ref[...], preferred_element_type=f32)
         - jnp.dot(k_ref[...], wk_ref[...], preferred_element_type=f32))
    v = jnp.dot(v_ref[...], wv_ref[...], preferred_element_type=f32)
    t_ref[...] = jnp.concatenate([d, v, pp_ref[...]], axis=1)


def _knn_kernel(crows_ref, ccols_ref, idx_ref, *, n, rb, k, boffset):
    cr = crows_ref[...]
    ca = ccols_ref[...]
    crb = cr.astype(jnp.bfloat16).astype(jnp.float32)
    cab = ca.astype(jnp.bfloat16).astype(jnp.float32)
    dot = lax.dot_general(crb, cab, (((1,), (0,)), ((), ())),
                          precision=lax.Precision.HIGHEST,
                          preferred_element_type=jnp.float32)
    d2r = jnp.sum(cr * cr, axis=1, keepdims=True)
    d2a = jnp.sum(ca * ca, axis=0, keepdims=True)
    dist = d2r + d2a - 2.0 * dot
    dshift = dist + 0.5
    seg = 256
    g_iota = lax.broadcasted_iota(jnp.int32, (rb, n), 1)
    vclass = lax.shift_right_logical(g_iota, 8)
    c = lax.bitcast_convert_type(
        (lax.bitcast_convert_type(dshift, jnp.int32) & jnp.int32(~0xF)) | vclass,
        jnp.float32)
    nseg = n // seg
    iotaseg = lax.broadcasted_iota(jnp.int32, (rb, seg), 1).astype(jnp.float32)
    removed = jnp.float32(jnp.inf)
    mprev = jnp.zeros((rb, 1), dtype=jnp.float32)
    cols = []
    for _ in range(k):
        s = jnp.where(c[:, 0:seg] > mprev, c[:, 0:seg], removed)
        for j in range(1, nseg):
            cj = c[:, j * seg:(j + 1) * seg]
            s = jnp.minimum(s, jnp.where(cj > mprev, cj, removed))
        m = jnp.min(s, axis=1, keepdims=True)
        lane = jnp.min(jnp.where(s == m, iotaseg, jnp.float32(seg)),
                       axis=1, keepdims=True)
        mb = lax.bitcast_convert_type(m, jnp.int32)
        am = ((mb & jnp.int32(15)) << 8) | lane.astype(jnp.int32)
        cols.append(am)
        mprev = m
    idxf = jnp.concatenate(cols, axis=1)
    idx_ref[...] = idxf + boffset


def _mlp_kernel(g_ref, pr_ref, p1_ref, pb1_ref, p2_ref, pb2_ref,
                a1_ref, ab1_ref, a2_ref, ab2_ref, wo_ref, bo_ref, out_ref, *, rb, k):
    f32 = jnp.float32
    g = g_ref[...]
    inner_w = p2_ref.shape[1]
    gd = g[:, 0:inner_w]
    gv = g[:, inner_w:2 * inner_w]
    gp = g[:, 2 * inner_w:2 * inner_w + PPAD]
    pc = pr_ref[...]
    rel = gp - jnp.broadcast_to(pc[:, None, :], (rb, k, PPAD)).reshape(rb * k, PPAD)
    h1 = jnp.maximum(jnp.dot(rel, p1_ref[...], preferred_element_type=f32)
                     + pb1_ref[...], 0.0)
    rpe = jnp.dot(h1, p2_ref[...], preferred_element_type=f32) + pb2_ref[...]
    s = jnp.dot(gd + rpe, a1_ref[...], preferred_element_type=f32) + ab1_ref[...]
    s = jnp.maximum(s, 0.0)
    sim = jnp.dot(s, a2_ref[...], preferred_element_type=f32) + ab2_ref[...]
    e = jnp.exp(sim)
    p = e / jnp.sum(e, axis=-1, keepdims=True)
    vg = gv + rpe
    inner = sim.shape[-1]
    p3 = p.reshape(rb, k, inner)
    vg3 = vg.reshape(rb, k, inner)
    num = jnp.sum(p3 * vg3, axis=1)
    ssq = jnp.sum(p3 * p3, axis=1)
    agg = num / jnp.maximum(jnp.sqrt(ssq), 1e-12)
    out_ref[...] = jnp.dot(agg, wo_ref[...], preferred_element_type=f32) + bo_ref[...]


def _make_gather(nrows, width):
    info = plsc.get_sparse_core_info()
    nc, ns = info.num_cores, info.num_subcores
    nw = nc * ns
    chunk = 128
    per_w = nrows // nw
    nchunk = per_w // chunk
    f32 = jnp.float32
    mesh = plsc.VectorSubcoreMesh(core_axis_name="c", subcore_axis_name="s")

    @functools.partial(
        pl.kernel, mesh=mesh,
        out_type=jax.ShapeDtypeStruct((nrows, width), f32),
        scratch_types=[pltpu.VMEM((chunk,), jnp.int32),
                       pltpu.VMEM((chunk, width), f32),
                       pltpu.SemaphoreType.DMA])
    def gather(tab, idx_hbm, out, idxv, buf, s_g):
        wid = lax.axis_index("s") * nc + lax.axis_index("c")
        base = wid * per_w

        def body(c, carry):
            off = base + c * chunk
            pltpu.sync_copy(idx_hbm.at[pl.ds(off, chunk)], idxv)
            pltpu.async_copy(tab.at[idxv], buf, s_g).wait()
            pltpu.sync_copy(buf, out.at[pl.ds(off, chunk)])
            return carry

        lax.fori_loop(0, nchunk, body, 0)

    return gather


def kernel(query, key, value, canonical, Wq, Wk, Wv, Wo, bo, P1, pb1, P2, pb2, A1, ab1, A2, ab2):
    bs, n, emb = query.shape
    inner = Wq.shape[0]
    ah = A1.shape[0]
    ph = P1.shape[0]
    k = K
    bn = bs * n
    f32 = jnp.float32

    qf = query.reshape(bn, emb)
    kf = key.reshape(bn, emb)
    vf = value.reshape(bn, emb)
    cf = canonical.reshape(bn, 3)

    width = 2 * inner + emb
    pospad = jnp.pad(cf, ((0, 0), (0, emb - 3)))
    rb1 = 512
    tab = pl.pallas_call(
        _proj_kernel,
        grid=(bn // rb1,),
        in_specs=[pl.BlockSpec((rb1, emb), lambda i: (i, 0))] * 4
        + [pl.BlockSpec((emb, inner), lambda i: (0, 0))] * 3,
        out_specs=pl.BlockSpec((rb1, width), lambda i: (i, 0)),
        out_shape=jax.ShapeDtypeStruct((bn, width), f32),
    )(qf, kf, vf, pospad, Wq.T, Wk.T, Wv.T)

    rb2 = 128
    nb2 = n // rb2
    ct = jnp.transpose(canonical, (0, 2, 1))
    pos16 = jnp.pad(cf, ((0, 0), (0, PPAD - 3)))
    p1ext = jnp.pad(P1.T, ((0, PPAD - 3), (0, 0)))
    rb4 = 128
    rows4 = rb4 * k
    gather_fn = _make_gather(n * k, width)

    outs = []
    for b in range(bs):
        idx_b = pl.pallas_call(
            functools.partial(_knn_kernel, n=n, rb=rb2, k=k, boffset=b * n),
            grid=(nb2,),
            in_specs=[pl.BlockSpec((rb2, 3), lambda i: (i, 0)),
                      pl.BlockSpec((3, n), lambda i: (0, 0))],
            out_specs=pl.BlockSpec((rb2, k), lambda i: (i, 0)),
            out_shape=jax.ShapeDtypeStruct((n, k), jnp.int32),
        )(canonical[b], ct[b])

        g_b = gather_fn(tab, idx_b.reshape(n * k))

        out_b = pl.pallas_call(
            functools.partial(_mlp_kernel, rb=rb4, k=k),
            grid=(n // rb4,),
            in_specs=[
                pl.BlockSpec((rows4, width), lambda i: (i, 0)),
                pl.BlockSpec((rb4, PPAD), lambda i: (i, 0)),
                pl.BlockSpec((PPAD, ph), lambda i: (0, 0)),
                pl.BlockSpec((1, ph), lambda i: (0, 0)),
                pl.BlockSpec((ph, inner), lambda i: (0, 0)),
                pl.BlockSpec((1, inner), lambda i: (0, 0)),
                pl.BlockSpec((inner, ah), lambda i: (0, 0)),
                pl.BlockSpec((1, ah), lambda i: (0, 0)),
                pl.BlockSpec((ah, inner), lambda i: (0, 0)),
                pl.BlockSpec((1, inner), lambda i: (0, 0)),
                pl.BlockSpec((inner, emb), lambda i: (0, 0)),
                pl.BlockSpec((1, emb), lambda i: (0, 0)),
            ],
            out_specs=pl.BlockSpec((rb4, emb), lambda i: (i, 0)),
            out_shape=jax.ShapeDtypeStruct((n, emb), f32),
        )(g_b, pos16[b * n:(b + 1) * n], p1ext, pb1.reshape(1, ph), P2.T, pb2.reshape(1, inner),
          A1.T, ab1.reshape(1, ah), A2.T, ab2.reshape(1, inner), Wo.T,
          bo.reshape(1, emb))
        outs.append(out_b)

    return jnp.stack(outs).reshape(bs, n, emb)
